# Initial kernel scaffold; baseline (speedup 1.0000x reference)
import functools

import jax
import jax.numpy as jnp
import numpy as np
from jax import lax
from jax.experimental import pallas as pl
from jax.experimental.pallas import tpu as pltpu
from jax.experimental.pallas import tpu_sc as plsc

B = 4096
L = 20
NTAB = 26
VOCAB = 100000
M = 32
NC = 2
NS = 16
NW = NC * NS
BAGS_W = B // NW
IDX_W = BAGS_W * L
NCHUNK = IDX_W // 128


def _sc_embed_body(idx_hbm, emb_hbm, out_hbm, idx_v, rows_v, out_v, sem):
    w = lax.axis_index("s") * NC + lax.axis_index("c")

    lane = lax.iota(jnp.int32, 16)

    def table_body(t, carry):
        tw = t * NW + w
        pltpu.sync_copy(idx_hbm.at[tw], idx_v)
        for c in range(NCHUNK):
            pltpu.async_copy(emb_hbm.at[idx_v.at[c]],
                             rows_v.at[pl.ds(c * 128, 128)], sem)
        for c in range(NCHUNK):
            pltpu.make_async_copy(emb_hbm.at[idx_v.at[0]],
                                  rows_v.at[pl.ds(c * 128, 128)], sem).wait()

        def bag_body(b, carry2):
            base = b * L
            acc0 = rows_v[base, pl.ds(0, 16)]
            acc1 = rows_v[base, pl.ds(16, 16)]
            for r in range(1, L):
                acc0 = acc0 + rows_v[base + r, pl.ds(0, 16)]
                acc1 = acc1 + rows_v[base + r, pl.ds(16, 16)]
            col = jnp.broadcast_to(b, (16,)).astype(jnp.int32)
            plsc.store_scatter(out_v, [lane, col], acc0)
            plsc.store_scatter(out_v, [lane + 16, col], acc1)
            return carry2

        lax.fori_loop(0, BAGS_W, bag_body, 0, unroll=False)
        pltpu.sync_copy(out_v,
                        out_hbm.at[pl.ds(t * M, M), pl.ds(w * BAGS_W, BAGS_W)])
        return carry

    lax.fori_loop(0, NTAB, table_body, 0, unroll=False)


def _sc_embed(idx3, emb_flat):
    mesh = plsc.VectorSubcoreMesh(core_axis_name="c", subcore_axis_name="s")
    kfn = pl.kernel(
        _sc_embed_body,
        out_type=jax.ShapeDtypeStruct((NTAB * M, B), jnp.float32),
        mesh=mesh,
        scratch_types=[
            pltpu.VMEM((NCHUNK, 128), jnp.int32),
            pltpu.VMEM((IDX_W, M), jnp.float32),
            pltpu.VMEM((M, BAGS_W), jnp.float32),
            pltpu.SemaphoreType.DMA,
        ],
    )
    return kfn(idx3, emb_flat)


def _tc_dense_body(dense_t, ly_t,
                   w0, b0, w1, b1, w2, b2,
                   tw0, tb0, tw1, tb1, tw2, tb2, out_ref):
    d = dense_t[...]
    x = jax.nn.relu(jnp.dot(w0[...], d, preferred_element_type=jnp.float32)
                    + b0[...][:, None])
    x = jax.nn.relu(jnp.dot(w1[...], x, preferred_element_type=jnp.float32)
                    + b1[...][:, None])
    x = jax.nn.relu(jnp.dot(w2[...], x, preferred_element_type=jnp.float32)
                    + b2[...][:, None])
    r = jnp.concatenate([x, ly_t[...]], axis=0)
    blk = r.shape[1]
    pieces = [x]
    for i in range(1, NTAB + 1):
        u = r[i * M:(i + 1) * M]
        r3 = r[:i * M].reshape(i, M, blk)
        pi = (r3 * u[None]).sum(axis=1)
        pieces.append(pi)
    z = jnp.concatenate(pieces, axis=0)
    h = jax.nn.relu(jnp.dot(tw0[...], z, preferred_element_type=jnp.float32)
                    + tb0[...][:, None])
    h = jax.nn.relu(jnp.dot(tw1[...], h, preferred_element_type=jnp.float32)
                    + tb1[...][:, None])
    p = jax.nn.sigmoid(jnp.dot(tw2[...], h, preferred_element_type=jnp.float32)
                       + tb2[...][:, None])
    out_ref[...] = p.T


def _tc_dense(dense_t, ly_t, weights):
    blk = 512
    grid = (B // blk,)
    full = lambda shape: pl.BlockSpec(shape, lambda i: (0,) * len(shape))
    in_specs = [
        pl.BlockSpec((13, blk), lambda i: (0, i)),
        pl.BlockSpec((NTAB * M, blk), lambda i: (0, i)),
    ] + [full(w.shape) for w in weights]
    out_specs = pl.BlockSpec((blk, 1), lambda i: (i, 0))
    return pl.pallas_call(
        _tc_dense_body,
        grid=grid,
        in_specs=in_specs,
        out_specs=out_specs,
        out_shape=jax.ShapeDtypeStruct((B, 1), jnp.float32),
    )(dense_t, ly_t, *weights)


def kernel(dense_x, lS_i, lS_o, emb_tables,
           bot_W0, bot_b0, bot_W1, bot_b1, bot_W2, bot_b2,
           top_W0, top_b0, top_W1, top_b1, top_W2, top_b2):
    del lS_o
    offs = (jnp.arange(NTAB, dtype=jnp.int32) * VOCAB)[:, None]
    idx3 = (lS_i + offs).reshape(NTAB, NW, NCHUNK, 128).reshape(
        NTAB * NW, NCHUNK, 128)
    emb_flat = emb_tables.reshape(NTAB * VOCAB, M)
    ly_t = _sc_embed(idx3, emb_flat)
    dense_t = dense_x.T
    weights = (bot_W0, bot_b0, bot_W1, bot_b1, bot_W2, bot_b2,
               top_W0, top_b0, top_W1, top_b1, top_W2, top_b2)
    return _tc_dense(dense_t, ly_t, weights)


# speedup vs baseline: 7.2492x; 7.2492x over previous
import functools

import jax
import jax.numpy as jnp
import numpy as np
from jax import lax
from jax.experimental import pallas as pl
from jax.experimental.pallas import tpu as pltpu
from jax.experimental.pallas import tpu_sc as plsc

B = 4096
L = 20
NTAB = 26
VOCAB = 100000
M = 32
NC = 2
NS = 16
NW = NC * NS
BAGS_W = B // NW
IDX_W = BAGS_W * L
NCHUNK = IDX_W // 128


def _sc_embed_body(idx_hbm, emb_hbm, out_hbm, idx_v, rows_v, out_v, sem):
    w = lax.axis_index("s") * NC + lax.axis_index("c")

    def table_body(t, carry):
        tw = t * NW + w
        pltpu.sync_copy(idx_hbm.at[tw], idx_v)
        for c in range(NCHUNK):
            pltpu.async_copy(emb_hbm.at[idx_v.at[c]],
                             rows_v.at[pl.ds(c * 128, 128)], sem)
        for c in range(NCHUNK):
            pltpu.make_async_copy(emb_hbm.at[idx_v.at[0]],
                                  rows_v.at[pl.ds(c * 128, 128)], sem).wait()

        def bag_body(b, carry2):
            base = b * L
            acc0 = rows_v[base, pl.ds(0, 16)]
            acc1 = rows_v[base, pl.ds(16, 16)]
            for r in range(1, L):
                acc0 = acc0 + rows_v[base + r, pl.ds(0, 16)]
                acc1 = acc1 + rows_v[base + r, pl.ds(16, 16)]
            out_v[b, pl.ds(0, 16)] = acc0
            out_v[b, pl.ds(16, 16)] = acc1
            return carry2

        lax.fori_loop(0, BAGS_W, bag_body, 0, unroll=False)
        pltpu.sync_copy(out_v,
                        out_hbm.at[t, pl.ds(w * BAGS_W, BAGS_W), :])
        return carry

    lax.fori_loop(0, NTAB, table_body, 0, unroll=False)


def _sc_embed(idx3, emb_flat):
    mesh = plsc.VectorSubcoreMesh(core_axis_name="c", subcore_axis_name="s")
    kfn = pl.kernel(
        _sc_embed_body,
        out_type=jax.ShapeDtypeStruct((NTAB, B, M), jnp.float32),
        mesh=mesh,
        scratch_types=[
            pltpu.VMEM((NCHUNK, 128), jnp.int32),
            pltpu.VMEM((IDX_W, M), jnp.float32),
            pltpu.VMEM((BAGS_W, M), jnp.float32),
            pltpu.SemaphoreType.DMA,
        ],
        compiler_params=pltpu.CompilerParams(use_tc_tiling_on_sc=False),
    )
    return kfn(idx3, emb_flat)


def _tc_dense_body(dense_t, ly_bm,
                   w0, b0, w1, b1, w2, b2,
                   tw0, tb0, tw1, tb1, tw2, tb2, out_ref):
    d = dense_t[...]
    x = jax.nn.relu(jnp.dot(w0[...], d, preferred_element_type=jnp.float32)
                    + b0[...][:, None])
    x = jax.nn.relu(jnp.dot(w1[...], x, preferred_element_type=jnp.float32)
                    + b1[...][:, None])
    x = jax.nn.relu(jnp.dot(w2[...], x, preferred_element_type=jnp.float32)
                    + b2[...][:, None])
    ly3 = ly_bm[...]
    ly_t = jnp.transpose(ly3, (0, 2, 1)).reshape(NTAB * M, ly3.shape[1])
    r = jnp.concatenate([x, ly_t], axis=0)
    blk = r.shape[1]
    pieces = [x]
    for i in range(1, NTAB + 1):
        u = r[i * M:(i + 1) * M]
        r3 = r[:i * M].reshape(i, M, blk)
        pi = (r3 * u[None]).sum(axis=1)
        pieces.append(pi)
    z = jnp.concatenate(pieces, axis=0)
    h = jax.nn.relu(jnp.dot(tw0[...], z, preferred_element_type=jnp.float32)
                    + tb0[...][:, None])
    h = jax.nn.relu(jnp.dot(tw1[...], h, preferred_element_type=jnp.float32)
                    + tb1[...][:, None])
    p = jax.nn.sigmoid(jnp.dot(tw2[...], h, preferred_element_type=jnp.float32)
                       + tb2[...][:, None])
    out_ref[...] = p.T


def _tc_dense(dense_t, ly_bm, weights):
    blk = 512
    grid = (B // blk,)
    full = lambda shape: pl.BlockSpec(shape, lambda i: (0,) * len(shape))
    in_specs = [
        pl.BlockSpec((13, blk), lambda i: (0, i)),
        pl.BlockSpec((NTAB, blk, M), lambda i: (0, i, 0)),
    ] + [full(w.shape) for w in weights]
    out_specs = pl.BlockSpec((blk, 1), lambda i: (i, 0))
    return pl.pallas_call(
        _tc_dense_body,
        grid=grid,
        in_specs=in_specs,
        out_specs=out_specs,
        out_shape=jax.ShapeDtypeStruct((B, 1), jnp.float32),
    )(dense_t, ly_bm, *weights)


def kernel(dense_x, lS_i, lS_o, emb_tables,
           bot_W0, bot_b0, bot_W1, bot_b1, bot_W2, bot_b2,
           top_W0, top_b0, top_W1, top_b1, top_W2, top_b2):
    del lS_o
    offs = (jnp.arange(NTAB, dtype=jnp.int32) * VOCAB)[:, None]
    idx3 = (lS_i + offs).reshape(NTAB, NW, NCHUNK, 128).reshape(
        NTAB * NW, NCHUNK, 128)
    emb_flat = emb_tables.reshape(NTAB * VOCAB, M)
    ly_bm = _sc_embed(idx3, emb_flat)
    dense_t = dense_x.T
    weights = (bot_W0, bot_b0, bot_W1, bot_b1, bot_W2, bot_b2,
               top_W0, top_b0, top_W1, top_b1, top_W2, top_b2)
    return _tc_dense(dense_t, ly_bm, weights)


# speedup vs baseline: 10.4816x; 1.4459x over previous
import functools

import jax
import jax.numpy as jnp
import numpy as np
from jax import lax
from jax.experimental import pallas as pl
from jax.experimental.pallas import tpu as pltpu
from jax.experimental.pallas import tpu_sc as plsc

B = 4096
L = 20
NTAB = 26
VOCAB = 100000
M = 32
NC = 2
NS = 16
NW = NC * NS
BAGS_W = B // NW
IDX_W = BAGS_W * L
NCHUNK = IDX_W // 128


def _sc_embed_body(idx_hbm, emb_hbm, out_hbm, idx_v, rows_v, out_v, sem):
    w = lax.axis_index("s") * NC + lax.axis_index("c")

    def table_body(t, carry):
        tw = t * NW + w
        pltpu.sync_copy(idx_hbm.at[tw], idx_v)
        for c in range(NCHUNK):
            pltpu.async_copy(emb_hbm.at[idx_v.at[c]],
                             rows_v.at[pl.ds(c * 128, 128)], sem)
        for c in range(NCHUNK):
            pltpu.make_async_copy(emb_hbm.at[idx_v.at[0]],
                                  rows_v.at[pl.ds(c * 128, 128)], sem).wait()

        def bag_body(b, carry2):
            base = b * L
            acc0 = rows_v[base, pl.ds(0, 16)]
            acc1 = rows_v[base, pl.ds(16, 16)]
            for r in range(1, L):
                acc0 = acc0 + rows_v[base + r, pl.ds(0, 16)]
                acc1 = acc1 + rows_v[base + r, pl.ds(16, 16)]
            out_v[b, pl.ds(0, 16)] = acc0
            out_v[b, pl.ds(16, 16)] = acc1
            return carry2

        lax.fori_loop(0, BAGS_W, bag_body, 0, unroll=False)
        pltpu.sync_copy(out_v,
                        out_hbm.at[t, pl.ds(w * BAGS_W, BAGS_W), :])
        return carry

    lax.fori_loop(0, NTAB, table_body, 0, unroll=False)


def _sc_embed(idx3, emb_flat):
    mesh = plsc.VectorSubcoreMesh(core_axis_name="c", subcore_axis_name="s")
    kfn = pl.kernel(
        _sc_embed_body,
        out_type=jax.ShapeDtypeStruct((NTAB, B, M), jnp.float32),
        mesh=mesh,
        scratch_types=[
            pltpu.VMEM((NCHUNK, 128), jnp.int32),
            pltpu.VMEM((IDX_W, M), jnp.float32),
            pltpu.VMEM((BAGS_W, M), jnp.float32),
            pltpu.SemaphoreType.DMA,
        ],
        compiler_params=pltpu.CompilerParams(use_tc_tiling_on_sc=False),
    )
    return kfn(idx3, emb_flat)


JB = VOCAB
JQ = JB // 4


def _tc_transpose_body(src_ref, out_ref):
    x = src_ref[0]
    for dj in range(4):
        out_ref[:, dj * M:(dj + 1) * M] = x[:, dj * JQ:(dj + 1) * JQ].T


def _tc_transpose(emb_t):
    return pl.pallas_call(
        _tc_transpose_body,
        grid=(NTAB,),
        in_specs=[pl.BlockSpec((1, M, JB), lambda t: (t, 0, 0))],
        out_specs=pl.BlockSpec((JQ, 4 * M), lambda t: (t, 0)),
        out_shape=jax.ShapeDtypeStruct((NTAB * VOCAB // 4, 4 * M), jnp.float32),
        compiler_params=pltpu.CompilerParams(vmem_limit_bytes=100 * 1024 * 1024),
    )(emb_t)


def _permute_idx(idx):
    b, l = idx // JB, idx % JB
    return (b * JQ + l % JQ) * 4 + l // JQ


def _tc_dense_body(dense_t, ly_bm,
                   w0, b0, w1, b1, w2, b2,
                   tw0, tb0, tw1, tb1, tw2, tb2, out_ref):
    d = dense_t[...]
    x = jax.nn.relu(jnp.dot(w0[...], d, preferred_element_type=jnp.float32)
                    + b0[...][:, None])
    x = jax.nn.relu(jnp.dot(w1[...], x, preferred_element_type=jnp.float32)
                    + b1[...][:, None])
    x = jax.nn.relu(jnp.dot(w2[...], x, preferred_element_type=jnp.float32)
                    + b2[...][:, None])
    ly3 = ly_bm[...]
    ly_t = jnp.transpose(ly3, (0, 2, 1)).reshape(NTAB * M, ly3.shape[1])
    r = jnp.concatenate([x, ly_t], axis=0)
    blk = r.shape[1]
    pieces = [x]
    for i in range(1, NTAB + 1):
        u = r[i * M:(i + 1) * M]
        r3 = r[:i * M].reshape(i, M, blk)
        pi = (r3 * u[None]).sum(axis=1)
        pieces.append(pi)
    z = jnp.concatenate(pieces, axis=0)
    h = jax.nn.relu(jnp.dot(tw0[...], z, preferred_element_type=jnp.float32)
                    + tb0[...][:, None])
    h = jax.nn.relu(jnp.dot(tw1[...], h, preferred_element_type=jnp.float32)
                    + tb1[...][:, None])
    p = jax.nn.sigmoid(jnp.dot(tw2[...], h, preferred_element_type=jnp.float32)
                       + tb2[...][:, None])
    out_ref[...] = p.T


def _tc_dense(dense_t, ly_bm, weights):
    blk = 512
    grid = (B // blk,)
    full = lambda shape: pl.BlockSpec(shape, lambda i: (0,) * len(shape))
    in_specs = [
        pl.BlockSpec((13, blk), lambda i: (0, i)),
        pl.BlockSpec((NTAB, blk, M), lambda i: (0, i, 0)),
    ] + [full(w.shape) for w in weights]
    out_specs = pl.BlockSpec((blk, 1), lambda i: (i, 0))
    return pl.pallas_call(
        _tc_dense_body,
        grid=grid,
        in_specs=in_specs,
        out_specs=out_specs,
        out_shape=jax.ShapeDtypeStruct((B, 1), jnp.float32),
    )(dense_t, ly_bm, *weights)


def kernel(dense_x, lS_i, lS_o, emb_tables,
           bot_W0, bot_b0, bot_W1, bot_b1, bot_W2, bot_b2,
           top_W0, top_b0, top_W1, top_b1, top_W2, top_b2):
    del lS_o
    offs = (jnp.arange(NTAB, dtype=jnp.int32) * VOCAB)[:, None]
    idx3 = _permute_idx(lS_i) + offs
    idx3 = idx3.reshape(NTAB, NW, NCHUNK, 128).reshape(NTAB * NW, NCHUNK, 128)
    emb_t = jnp.transpose(emb_tables, (0, 2, 1))
    emb_flat = _tc_transpose(emb_t).reshape(NTAB * VOCAB, M)
    ly_bm = _sc_embed(idx3, emb_flat)
    dense_t = dense_x.T
    weights = (bot_W0, bot_b0, bot_W1, bot_b1, bot_W2, bot_b2,
               top_W0, top_b0, top_W1, top_b1, top_W2, top_b2)
    return _tc_dense(dense_t, ly_bm, weights)


# speedup vs baseline: 18.1563x; 1.7322x over previous
import functools

import jax
import jax.numpy as jnp
import numpy as np
from jax import lax
from jax.experimental import pallas as pl
from jax.experimental.pallas import tpu as pltpu
from jax.experimental.pallas import tpu_sc as plsc

B = 4096
L = 20
NTAB = 26
VOCAB = 100000
M = 32
NC = 2
NS = 16
NW = NC * NS
BAGS_W = B // NW
IDX_W = BAGS_W * L
NCHUNK = IDX_W // 128


def _sc_embed_body(idx_hbm, emb_hbm, out_hbm, idx_v, rows_v, out_v, sem):
    w = lax.axis_index("s") * NC + lax.axis_index("c")

    def table_body(t, carry):
        tw = t * NW + w
        pltpu.sync_copy(idx_hbm.at[tw], idx_v)
        for c in range(NCHUNK):
            pltpu.async_copy(emb_hbm.at[idx_v.at[c]],
                             rows_v.at[pl.ds(c * 128, 128)], sem)
        for c in range(NCHUNK):
            pltpu.make_async_copy(emb_hbm.at[idx_v.at[0]],
                                  rows_v.at[pl.ds(c * 128, 128)], sem).wait()

        def bag_body(b, carry2):
            base = b * L
            acc0 = rows_v[base, pl.ds(0, 16)]
            acc1 = rows_v[base, pl.ds(16, 16)]
            for r in range(1, L):
                acc0 = acc0 + rows_v[base + r, pl.ds(0, 16)]
                acc1 = acc1 + rows_v[base + r, pl.ds(16, 16)]
            out_v[b, pl.ds(0, 16)] = acc0
            out_v[b, pl.ds(16, 16)] = acc1
            return carry2

        lax.fori_loop(0, BAGS_W, bag_body, 0, unroll=False)
        pltpu.sync_copy(out_v,
                        out_hbm.at[t, pl.ds(w * BAGS_W, BAGS_W), :])
        return carry

    lax.fori_loop(0, NTAB, table_body, 0, unroll=False)


def _sc_embed(idx3, emb_flat):
    mesh = plsc.VectorSubcoreMesh(core_axis_name="c", subcore_axis_name="s")
    kfn = pl.kernel(
        _sc_embed_body,
        out_type=jax.ShapeDtypeStruct((NTAB, B, M), jnp.float32),
        mesh=mesh,
        scratch_types=[
            pltpu.VMEM((NCHUNK, 128), jnp.int32),
            pltpu.VMEM((IDX_W, M), jnp.float32),
            pltpu.VMEM((BAGS_W, M), jnp.float32),
            pltpu.SemaphoreType.DMA,
        ],
        compiler_params=pltpu.CompilerParams(use_tc_tiling_on_sc=False),
    )
    return kfn(idx3, emb_flat)


JB = VOCAB
JQ = JB // 4


def _tc_transpose_body(src_ref, out_ref):
    x = src_ref[0]
    ch = JQ // 8
    for c in range(8):
        xs = jnp.concatenate(
            [x[:, dj * JQ + c * ch:dj * JQ + (c + 1) * ch] for dj in range(4)],
            axis=0)
        out_ref[c * ch:(c + 1) * ch, :] = xs.T


def _tc_transpose(emb_t):
    return pl.pallas_call(
        _tc_transpose_body,
        grid=(NTAB,),
        in_specs=[pl.BlockSpec((1, M, JB), lambda t: (t, 0, 0))],
        out_specs=pl.BlockSpec((JQ, 4 * M), lambda t: (t, 0)),
        out_shape=jax.ShapeDtypeStruct((NTAB * VOCAB // 4, 4 * M), jnp.float32),
        compiler_params=pltpu.CompilerParams(vmem_limit_bytes=100 * 1024 * 1024),
    )(emb_t)


def _permute_idx(idx):
    b, l = idx // JB, idx % JB
    return (b * JQ + l % JQ) * 4 + l // JQ


def _tc_dense_body(dense_t, ly_bm,
                   w0, b0, w1, b1, w2, b2,
                   tw0, tb0, tw1, tb1, tw2, tb2, out_ref):
    d = dense_t[...]
    x = jax.nn.relu(jnp.dot(w0[...], d, preferred_element_type=jnp.float32)
                    + b0[...][:, None])
    x = jax.nn.relu(jnp.dot(w1[...], x, preferred_element_type=jnp.float32)
                    + b1[...][:, None])
    x = jax.nn.relu(jnp.dot(w2[...], x, preferred_element_type=jnp.float32)
                    + b2[...][:, None])
    ly3 = ly_bm[...]
    ly_t = jnp.transpose(ly3, (0, 2, 1)).reshape(NTAB * M, ly3.shape[1])
    r = jnp.concatenate([x, ly_t], axis=0)
    blk = r.shape[1]
    pieces = [x]
    for i in range(1, NTAB + 1):
        u = r[i * M:(i + 1) * M]
        r3 = r[:i * M].reshape(i, M, blk)
        pi = (r3 * u[None]).sum(axis=1)
        pieces.append(pi)
    z = jnp.concatenate(pieces, axis=0)
    h = jax.nn.relu(jnp.dot(tw0[...], z, preferred_element_type=jnp.float32)
                    + tb0[...][:, None])
    h = jax.nn.relu(jnp.dot(tw1[...], h, preferred_element_type=jnp.float32)
                    + tb1[...][:, None])
    p = jax.nn.sigmoid(jnp.dot(tw2[...], h, preferred_element_type=jnp.float32)
                       + tb2[...][:, None])
    out_ref[...] = p.T


def _tc_dense(dense_t, ly_bm, weights):
    blk = 512
    grid = (B // blk,)
    full = lambda shape: pl.BlockSpec(shape, lambda i: (0,) * len(shape))
    in_specs = [
        pl.BlockSpec((13, blk), lambda i: (0, i)),
        pl.BlockSpec((NTAB, blk, M), lambda i: (0, i, 0)),
    ] + [full(w.shape) for w in weights]
    out_specs = pl.BlockSpec((blk, 1), lambda i: (i, 0))
    return pl.pallas_call(
        _tc_dense_body,
        grid=grid,
        in_specs=in_specs,
        out_specs=out_specs,
        out_shape=jax.ShapeDtypeStruct((B, 1), jnp.float32),
    )(dense_t, ly_bm, *weights)


def kernel(dense_x, lS_i, lS_o, emb_tables,
           bot_W0, bot_b0, bot_W1, bot_b1, bot_W2, bot_b2,
           top_W0, top_b0, top_W1, top_b1, top_W2, top_b2):
    del lS_o
    offs = (jnp.arange(NTAB, dtype=jnp.int32) * VOCAB)[:, None]
    idx3 = _permute_idx(lS_i) + offs
    idx3 = idx3.reshape(NTAB, NW, NCHUNK, 128).reshape(NTAB * NW, NCHUNK, 128)
    emb_t = jnp.transpose(emb_tables, (0, 2, 1))
    emb_flat = _tc_transpose(emb_t).reshape(NTAB * VOCAB, M)
    ly_bm = _sc_embed(idx3, emb_flat)
    dense_t = dense_x.T
    weights = (bot_W0, bot_b0, bot_W1, bot_b1, bot_W2, bot_b2,
               top_W0, top_b0, top_W1, top_b1, top_W2, top_b2)
    return _tc_dense(dense_t, ly_bm, weights)
